# Initial kernel scaffold; baseline (speedup 1.0000x reference)
import jax
import jax.numpy as jnp
from jax.experimental import pallas as pl


def kernel(feat, edge_index, W1, b1, W2, b2):
    raise NotImplementedError("write your pallas kernel here")


# speedup vs baseline: 8.0112x; 8.0112x over previous
import functools

import jax
import jax.numpy as jnp
from jax import lax
from jax.experimental import pallas as pl
from jax.experimental.pallas import tpu as pltpu
from jax.experimental.pallas import tpu_sc as plsc

_N = 10000
_E = 320000
_D1 = 128
_D2 = 48
_NCLS = 40

_NC = 2
_NS = 16
_NW = _NC * _NS
_NPAD = 10240
_RPT = _NPAD // _NS
_EPT = _E // _NW
_KD = 2000
_K = 200

_mesh = plsc.VectorSubcoreMesh(core_axis_name="c", subcore_axis_name="s")


def _deg_body(src_h, dst_h, ones_h, zed_h, out_h,
              idx_v, ones_v, dego_sp, degi_sp):
    cid = lax.axis_index("c")
    sid = lax.axis_index("s")
    wid = cid * _NS + sid
    r0 = sid * _RPT
    pltpu.sync_copy(zed_h.at[pl.ds(r0, _RPT)], dego_sp.at[pl.ds(r0, _RPT)])
    pltpu.sync_copy(zed_h.at[pl.ds(r0, _RPT)], degi_sp.at[pl.ds(r0, _RPT)])
    pltpu.sync_copy(ones_h, ones_v)
    plsc.subcore_barrier()
    ebase = wid * _EPT

    def body(j, carry):
        b = ebase + j * _KD
        pltpu.sync_copy(src_h.at[pl.ds(b, _KD)], idx_v)
        pltpu.sync_copy(ones_v, dego_sp.at[idx_v], add=True)
        pltpu.sync_copy(dst_h.at[pl.ds(b, _KD)], idx_v)
        pltpu.sync_copy(ones_v, degi_sp.at[idx_v], add=True)
        return carry

    lax.fori_loop(0, _EPT // _KD, body, 0)
    plsc.subcore_barrier()
    pltpu.sync_copy(dego_sp.at[pl.ds(r0, _RPT)],
                    out_h.at[pl.ds(cid * _NPAD + r0, _RPT)])
    pltpu.sync_copy(degi_sp.at[pl.ds(r0, _RPT)],
                    out_h.at[pl.ds((2 + cid) * _NPAD + r0, _RPT)])


_deg_call = pl.kernel(
    _deg_body,
    mesh=_mesh,
    out_type=jax.ShapeDtypeStruct((4 * _NPAD,), jnp.float32),
    scratch_types=[
        pltpu.VMEM((_KD,), jnp.int32),
        pltpu.VMEM((_KD,), jnp.float32),
        pltpu.VMEM_SHARED((_NPAD,), jnp.float32),
        pltpu.VMEM_SHARED((_NPAD,), jnp.float32),
    ],
)


def _seg_body(h_h, src_h, dst_h, zed_h, out_h,
              sidx_v, didx_v, rows_v, acc_sp, sem):
    cid = lax.axis_index("c")
    sid = lax.axis_index("s")
    wid = cid * _NS + sid
    r0 = sid * _RPT
    pltpu.sync_copy(zed_h.at[pl.ds(r0, _RPT)], acc_sp.at[pl.ds(r0, _RPT)])
    plsc.subcore_barrier()
    ebase = wid * _EPT

    def body(j, carry):
        b = ebase + j * _K
        pltpu.sync_copy(src_h.at[pl.ds(b, _K)], sidx_v)
        pltpu.sync_copy(dst_h.at[pl.ds(b, _K)], didx_v)
        pltpu.async_copy(h_h.at[sidx_v], rows_v, sem).wait()
        pltpu.sync_copy(rows_v, acc_sp.at[didx_v], add=True)
        return carry

    lax.fori_loop(0, _EPT // _K, body, 0)
    plsc.subcore_barrier()
    pltpu.sync_copy(acc_sp.at[pl.ds(r0, _RPT)],
                    out_h.at[pl.ds(cid * _NPAD + r0, _RPT)])


def _make_seg(d):
    return pl.kernel(
        _seg_body,
        mesh=_mesh,
        out_type=jax.ShapeDtypeStruct((2 * _NPAD, d), jnp.float32),
        scratch_types=[
            pltpu.VMEM((_K,), jnp.int32),
            pltpu.VMEM((_K,), jnp.int32),
            pltpu.VMEM((_K, d), jnp.float32),
            pltpu.VMEM_SHARED((_NPAD, d), jnp.float32),
            pltpu.SemaphoreType.DMA,
        ],
        compiler_params=pltpu.CompilerParams(use_tc_tiling_on_sc=False),
    )


_seg_d1 = _make_seg(_D1)
_seg_d2 = _make_seg(_D2)


def _h1_body(deg_ref, feat_ref, h1_ref):
    deg_out = deg_ref[:, 0:1] + deg_ref[:, 1:2]
    norm_src = lax.rsqrt(jnp.maximum(deg_out, 1.0))
    h1_ref[...] = feat_ref[...] * norm_src


def _mid_body(a0_ref, a1_ref, deg_ref, w1_ref, b1_ref, w2_ref, h2_ref):
    deg_out = deg_ref[:, 0:1] + deg_ref[:, 1:2]
    deg_in = deg_ref[:, 2:3] + deg_ref[:, 3:4]
    norm_src = lax.rsqrt(jnp.maximum(deg_out, 1.0))
    norm_dst = lax.rsqrt(jnp.maximum(deg_in, 1.0))
    agg = a0_ref[...] + a1_ref[...]
    x1 = jnp.dot(agg, w1_ref[...], preferred_element_type=jnp.float32)
    x1 = jnp.maximum(x1 * norm_dst + b1_ref[...][None, :], 0.0)
    h2_ref[...] = jnp.dot(x1 * norm_src, w2_ref[...],
                          preferred_element_type=jnp.float32)


def _fin_body(a0_ref, a1_ref, deg_ref, b2_ref, out_ref):
    deg_in = deg_ref[:, 2:3] + deg_ref[:, 3:4]
    norm_dst = lax.rsqrt(jnp.maximum(deg_in, 1.0))
    agg = a0_ref[...] + a1_ref[...]
    out_ref[...] = agg[:, :_NCLS] * norm_dst + b2_ref[...][None, :]


def kernel(feat, edge_index, W1, b1, W2, b2):
    src = edge_index[0].astype(jnp.int32)
    dst = edge_index[1].astype(jnp.int32)

    ones = jnp.ones((_KD,), jnp.float32)
    zed1d = jnp.zeros((_NPAD,), jnp.float32)
    deg4 = _deg_call(src, dst, ones, zed1d)
    degt = jnp.transpose(deg4.reshape(4, _NPAD))[:_N]

    h1 = pl.pallas_call(
        _h1_body,
        out_shape=jax.ShapeDtypeStruct((_N, _D1), jnp.float32),
    )(degt, feat)

    zed1 = jnp.zeros((_NPAD, _D1), jnp.float32)
    aggp = _seg_d1(h1, src, dst, zed1)
    a0 = aggp[:_N]
    a1 = aggp[_NPAD:_NPAD + _N]

    w2p = jnp.pad(W2, ((0, 0), (0, _D2 - _NCLS)))
    h2 = pl.pallas_call(
        _mid_body,
        out_shape=jax.ShapeDtypeStruct((_N, _D2), jnp.float32),
    )(a0, a1, degt, W1, b1, w2p)

    zed2b = jnp.zeros((_NPAD, _D2), jnp.float32)
    agg2p = _seg_d2(h2, src, dst, zed2b)

    out = pl.pallas_call(
        _fin_body,
        out_shape=jax.ShapeDtypeStruct((_N, _NCLS), jnp.float32),
    )(agg2p[:_N], agg2p[_NPAD:_NPAD + _N], degt, b2)
    return out


# speedup vs baseline: 12.6251x; 1.5759x over previous
import functools

import jax
import jax.numpy as jnp
from jax import lax
from jax.experimental import pallas as pl
from jax.experimental.pallas import tpu as pltpu
from jax.experimental.pallas import tpu_sc as plsc

_N = 10000
_E = 320000
_D1 = 128
_D2 = 48
_NCLS = 40

_NC = 2
_NS = 16
_NW = _NC * _NS
_NPAD = 10240
_RPT = _NPAD // _NS
_EPT = _E // _NW
_KD = 2000
_K = 200

_mesh = plsc.VectorSubcoreMesh(core_axis_name="c", subcore_axis_name="s")


def _deg_body(src_h, dst_h, ones_h, zed_h, out_h,
              idx_v, ones_v, dego_sp, degi_sp):
    cid = lax.axis_index("c")
    sid = lax.axis_index("s")
    wid = cid * _NS + sid
    r0 = sid * _RPT
    pltpu.sync_copy(zed_h.at[pl.ds(r0, _RPT)], dego_sp.at[pl.ds(r0, _RPT)])
    pltpu.sync_copy(zed_h.at[pl.ds(r0, _RPT)], degi_sp.at[pl.ds(r0, _RPT)])
    pltpu.sync_copy(ones_h, ones_v)
    plsc.subcore_barrier()
    ebase = wid * _EPT

    def body(j, carry):
        b = ebase + j * _KD
        pltpu.sync_copy(src_h.at[pl.ds(b, _KD)], idx_v)
        pltpu.sync_copy(ones_v, dego_sp.at[idx_v], add=True)
        pltpu.sync_copy(dst_h.at[pl.ds(b, _KD)], idx_v)
        pltpu.sync_copy(ones_v, degi_sp.at[idx_v], add=True)
        return carry

    lax.fori_loop(0, _EPT // _KD, body, 0)
    plsc.subcore_barrier()
    pltpu.sync_copy(dego_sp.at[pl.ds(r0, _RPT)],
                    out_h.at[pl.ds(cid * _NPAD + r0, _RPT)])
    pltpu.sync_copy(degi_sp.at[pl.ds(r0, _RPT)],
                    out_h.at[pl.ds((2 + cid) * _NPAD + r0, _RPT)])


_deg_call = pl.kernel(
    _deg_body,
    mesh=_mesh,
    out_type=jax.ShapeDtypeStruct((4 * _NPAD,), jnp.float32),
    scratch_types=[
        pltpu.VMEM((_KD,), jnp.int32),
        pltpu.VMEM((_KD,), jnp.float32),
        pltpu.VMEM_SHARED((_NPAD,), jnp.float32),
        pltpu.VMEM_SHARED((_NPAD,), jnp.float32),
    ],
)


def _seg_body(k, nch, h_h, src3_h, dst3_h, zed_h, out_h,
              sidx_v, didx_v, rows0_v, rows1_v, acc_sp, g0, g1):
    cid = lax.axis_index("c")
    sid = lax.axis_index("s")
    wid = cid * _NS + sid
    r0 = sid * _RPT
    pltpu.sync_copy(zed_h.at[pl.ds(r0, _RPT)], acc_sp.at[pl.ds(r0, _RPT)])
    pltpu.sync_copy(src3_h.at[wid], sidx_v)
    pltpu.sync_copy(dst3_h.at[wid], didx_v)
    plsc.subcore_barrier()

    dummy = h_h.at[pl.ds(0, k)]
    pltpu.async_copy(h_h.at[sidx_v.at[0]], rows0_v, g0)

    def body(i, carry):
        j0 = 2 * i
        pltpu.async_copy(h_h.at[sidx_v.at[j0 + 1]], rows1_v, g1)
        pltpu.make_async_copy(dummy, rows0_v, g0).wait()
        pltpu.sync_copy(rows0_v, acc_sp.at[didx_v.at[j0]], add=True)
        pltpu.async_copy(h_h.at[sidx_v.at[j0 + 2]], rows0_v, g0)
        pltpu.make_async_copy(dummy, rows1_v, g1).wait()
        pltpu.sync_copy(rows1_v, acc_sp.at[didx_v.at[j0 + 1]], add=True)
        return carry

    lax.fori_loop(0, (nch - 1) // 2, body, 0)
    pltpu.make_async_copy(dummy, rows0_v, g0).wait()
    pltpu.sync_copy(rows0_v, acc_sp.at[didx_v.at[nch - 1]], add=True)
    plsc.subcore_barrier()
    pltpu.sync_copy(acc_sp.at[pl.ds(r0, _RPT)],
                    out_h.at[pl.ds(cid * _NPAD + r0, _RPT)])


def _make_seg(d, k):
    nch = _EPT // k
    assert nch % 2 == 1 and k % 8 == 0
    return pl.kernel(
        functools.partial(_seg_body, k, nch),
        mesh=_mesh,
        out_type=jax.ShapeDtypeStruct((2 * _NPAD, d), jnp.float32),
        scratch_types=[
            pltpu.VMEM((nch, k), jnp.int32),
            pltpu.VMEM((nch, k), jnp.int32),
            pltpu.VMEM((k, d), jnp.float32),
            pltpu.VMEM((k, d), jnp.float32),
            pltpu.VMEM_SHARED((_NPAD, d), jnp.float32),
            pltpu.SemaphoreType.DMA,
            pltpu.SemaphoreType.DMA,
        ],
        compiler_params=pltpu.CompilerParams(use_tc_tiling_on_sc=False),
    )


_K1 = 80
_K2 = 400
_seg_d1 = _make_seg(_D1, _K1)
_seg_d2 = _make_seg(_D2, _K2)


def _h1_body(deg_ref, feat_ref, h1_ref):
    deg_out = deg_ref[:, 0:1] + deg_ref[:, 1:2]
    norm_src = lax.rsqrt(jnp.maximum(deg_out, 1.0))
    h1_ref[...] = feat_ref[...] * norm_src


def _mid_body(a0_ref, a1_ref, deg_ref, w1_ref, b1_ref, w2_ref, h2_ref):
    deg_out = deg_ref[:, 0:1] + deg_ref[:, 1:2]
    deg_in = deg_ref[:, 2:3] + deg_ref[:, 3:4]
    norm_src = lax.rsqrt(jnp.maximum(deg_out, 1.0))
    norm_dst = lax.rsqrt(jnp.maximum(deg_in, 1.0))
    agg = a0_ref[...] + a1_ref[...]
    x1 = jnp.dot(agg, w1_ref[...], preferred_element_type=jnp.float32)
    x1 = jnp.maximum(x1 * norm_dst + b1_ref[...][None, :], 0.0)
    h2_ref[...] = jnp.dot(x1 * norm_src, w2_ref[...],
                          preferred_element_type=jnp.float32)


def _fin_body(a0_ref, a1_ref, deg_ref, b2_ref, out_ref):
    deg_in = deg_ref[:, 2:3] + deg_ref[:, 3:4]
    norm_dst = lax.rsqrt(jnp.maximum(deg_in, 1.0))
    agg = a0_ref[...] + a1_ref[...]
    out_ref[...] = agg[:, :_NCLS] * norm_dst + b2_ref[...][None, :]


def kernel(feat, edge_index, W1, b1, W2, b2):
    src = edge_index[0].astype(jnp.int32)
    dst = edge_index[1].astype(jnp.int32)

    ones = jnp.ones((_KD,), jnp.float32)
    zed1d = jnp.zeros((_NPAD,), jnp.float32)
    deg4 = _deg_call(src, dst, ones, zed1d)
    degt = jnp.transpose(deg4.reshape(4, _NPAD))[:_N]

    h1 = pl.pallas_call(
        _h1_body,
        out_shape=jax.ShapeDtypeStruct((_N, _D1), jnp.float32),
    )(degt, feat)

    zed1 = jnp.zeros((_NPAD, _D1), jnp.float32)
    src1 = src.reshape(_NW, _EPT // _K1, _K1)
    dst1 = dst.reshape(_NW, _EPT // _K1, _K1)
    aggp = _seg_d1(h1, src1, dst1, zed1)
    a0 = aggp[:_N]
    a1 = aggp[_NPAD:_NPAD + _N]

    w2p = jnp.pad(W2, ((0, 0), (0, _D2 - _NCLS)))
    h2 = pl.pallas_call(
        _mid_body,
        out_shape=jax.ShapeDtypeStruct((_N, _D2), jnp.float32),
    )(a0, a1, degt, W1, b1, w2p)

    zed2b = jnp.zeros((_NPAD, _D2), jnp.float32)
    src2 = src.reshape(_NW, _EPT // _K2, _K2)
    dst2 = dst.reshape(_NW, _EPT // _K2, _K2)
    agg2p = _seg_d2(h2, src2, dst2, zed2b)

    out = pl.pallas_call(
        _fin_body,
        out_shape=jax.ShapeDtypeStruct((_N, _NCLS), jnp.float32),
    )(agg2p[:_N], agg2p[_NPAD:_NPAD + _N], degt, b2)
    return out


# speedup vs baseline: 13.2232x; 1.0474x over previous
import functools

import jax
import jax.numpy as jnp
from jax import lax
from jax.experimental import pallas as pl
from jax.experimental.pallas import tpu as pltpu
from jax.experimental.pallas import tpu_sc as plsc

_N = 10000
_E = 320000
_D1 = 128
_D2 = 48
_NCLS = 40

_NC = 2
_NS = 16
_NW = _NC * _NS
_NPAD = 10240
_RPT = _NPAD // _NS
_EPT = _E // _NW
_KD = 2000
_K = 200

_mesh = plsc.VectorSubcoreMesh(core_axis_name="c", subcore_axis_name="s")


def _deg_body(src_h, dst_h, ones_h, zed_h, out_h,
              idx_v, ones_v, dego_sp, degi_sp):
    cid = lax.axis_index("c")
    sid = lax.axis_index("s")
    wid = cid * _NS + sid
    r0 = sid * _RPT
    pltpu.sync_copy(zed_h.at[pl.ds(r0, _RPT)], dego_sp.at[pl.ds(r0, _RPT)])
    pltpu.sync_copy(zed_h.at[pl.ds(r0, _RPT)], degi_sp.at[pl.ds(r0, _RPT)])
    pltpu.sync_copy(ones_h, ones_v)
    plsc.subcore_barrier()
    ebase = wid * _EPT

    def body(j, carry):
        b = ebase + j * _KD
        pltpu.sync_copy(src_h.at[pl.ds(b, _KD)], idx_v)
        pltpu.sync_copy(ones_v, dego_sp.at[idx_v], add=True)
        pltpu.sync_copy(dst_h.at[pl.ds(b, _KD)], idx_v)
        pltpu.sync_copy(ones_v, degi_sp.at[idx_v], add=True)
        return carry

    lax.fori_loop(0, _EPT // _KD, body, 0)
    plsc.subcore_barrier()
    pltpu.sync_copy(dego_sp.at[pl.ds(r0, _RPT)],
                    out_h.at[pl.ds(cid * _NPAD + r0, _RPT)])
    pltpu.sync_copy(degi_sp.at[pl.ds(r0, _RPT)],
                    out_h.at[pl.ds((2 + cid) * _NPAD + r0, _RPT)])


_deg_call = pl.kernel(
    _deg_body,
    mesh=_mesh,
    out_type=jax.ShapeDtypeStruct((4 * _NPAD,), jnp.float32),
    scratch_types=[
        pltpu.VMEM((_KD,), jnp.int32),
        pltpu.VMEM((_KD,), jnp.float32),
        pltpu.VMEM_SHARED((_NPAD,), jnp.float32),
        pltpu.VMEM_SHARED((_NPAD,), jnp.float32),
    ],
)


def _seg_body(k, nch, h_h, src3_h, dst3_h, zed_h, out_h,
              sidx_v, didx_v, rows0_v, rows1_v, acc_sp, g0, g1):
    cid = lax.axis_index("c")
    sid = lax.axis_index("s")
    wid = cid * _NS + sid
    r0 = sid * _RPT
    pltpu.sync_copy(zed_h.at[pl.ds(r0, _RPT)], acc_sp.at[pl.ds(r0, _RPT)])
    pltpu.sync_copy(src3_h.at[wid], sidx_v)
    pltpu.sync_copy(dst3_h.at[wid], didx_v)
    plsc.subcore_barrier()

    dummy = h_h.at[pl.ds(0, k)]
    pltpu.async_copy(h_h.at[sidx_v.at[0]], rows0_v, g0)

    def body(i, carry):
        j0 = 2 * i
        pltpu.async_copy(h_h.at[sidx_v.at[j0 + 1]], rows1_v, g1)
        pltpu.make_async_copy(dummy, rows0_v, g0).wait()
        pltpu.sync_copy(rows0_v, acc_sp.at[didx_v.at[j0]], add=True)
        pltpu.async_copy(h_h.at[sidx_v.at[j0 + 2]], rows0_v, g0)
        pltpu.make_async_copy(dummy, rows1_v, g1).wait()
        pltpu.sync_copy(rows1_v, acc_sp.at[didx_v.at[j0 + 1]], add=True)
        return carry

    lax.fori_loop(0, (nch - 1) // 2, body, 0)
    pltpu.make_async_copy(dummy, rows0_v, g0).wait()
    pltpu.sync_copy(rows0_v, acc_sp.at[didx_v.at[nch - 1]], add=True)
    plsc.subcore_barrier()
    pltpu.sync_copy(acc_sp.at[pl.ds(r0, _RPT)],
                    out_h.at[pl.ds(cid * _NPAD + r0, _RPT)])


def _make_seg(d, k):
    nch = _EPT // k
    assert nch % 2 == 1 and k % 8 == 0
    return pl.kernel(
        functools.partial(_seg_body, k, nch),
        mesh=_mesh,
        out_type=jax.ShapeDtypeStruct((2 * _NPAD, d), jnp.float32),
        scratch_types=[
            pltpu.VMEM((nch, k), jnp.int32),
            pltpu.VMEM((nch, k), jnp.int32),
            pltpu.VMEM((k, d), jnp.float32),
            pltpu.VMEM((k, d), jnp.float32),
            pltpu.VMEM_SHARED((_NPAD, d), jnp.float32),
            pltpu.SemaphoreType.DMA,
            pltpu.SemaphoreType.DMA,
        ],
        compiler_params=pltpu.CompilerParams(use_tc_tiling_on_sc=False),
    )


_K1 = 80
_K2 = 400
_seg_d1 = _make_seg(_D1, _K1)
_seg_d2 = _make_seg(_D2, _K2)


def _h1_body(deg_ref, feat_ref, h1_ref):
    d = deg_ref[pl.ds(0, _N), :]
    deg_out = d[:, 0:1] + d[:, 1:2]
    norm_src = lax.rsqrt(jnp.maximum(deg_out, 1.0))
    h1_ref[...] = feat_ref[...] * norm_src


def _mid_body(aggp_ref, deg_ref, w1_ref, b1_ref, w2_ref, h2_ref):
    d = deg_ref[pl.ds(0, _N), :]
    deg_out = d[:, 0:1] + d[:, 1:2]
    deg_in = d[:, 2:3] + d[:, 3:4]
    norm_src = lax.rsqrt(jnp.maximum(deg_out, 1.0))
    norm_dst = lax.rsqrt(jnp.maximum(deg_in, 1.0))
    agg = aggp_ref[pl.ds(0, _N), :] + aggp_ref[pl.ds(_NPAD, _N), :]
    x1 = jnp.dot(agg, w1_ref[...], preferred_element_type=jnp.float32)
    x1 = jnp.maximum(x1 * norm_dst + b1_ref[...][None, :], 0.0)
    h2_ref[...] = jnp.dot(x1 * norm_src, w2_ref[...],
                          preferred_element_type=jnp.float32)


def _fin_body(aggp_ref, deg_ref, b2_ref, out_ref):
    d = deg_ref[pl.ds(0, _N), :]
    deg_in = d[:, 2:3] + d[:, 3:4]
    norm_dst = lax.rsqrt(jnp.maximum(deg_in, 1.0))
    agg = aggp_ref[pl.ds(0, _N), :] + aggp_ref[pl.ds(_NPAD, _N), :]
    out_ref[...] = agg[:, :_NCLS] * norm_dst + b2_ref[...][None, :]


def kernel(feat, edge_index, W1, b1, W2, b2):
    src = edge_index[0].astype(jnp.int32)
    dst = edge_index[1].astype(jnp.int32)

    ones = jnp.ones((_KD,), jnp.float32)
    zed1d = jnp.zeros((_NPAD,), jnp.float32)
    deg4 = _deg_call(src, dst, ones, zed1d)
    degt = jnp.transpose(deg4.reshape(4, _NPAD))

    h1 = pl.pallas_call(
        _h1_body,
        out_shape=jax.ShapeDtypeStruct((_N, _D1), jnp.float32),
    )(degt, feat)

    zed1 = jnp.zeros((_NPAD, _D1), jnp.float32)
    src1 = src.reshape(_NW, _EPT // _K1, _K1)
    dst1 = dst.reshape(_NW, _EPT // _K1, _K1)
    aggp = _seg_d1(h1, src1, dst1, zed1)

    w2p = jnp.pad(W2, ((0, 0), (0, _D2 - _NCLS)))
    h2 = pl.pallas_call(
        _mid_body,
        out_shape=jax.ShapeDtypeStruct((_N, _D2), jnp.float32),
    )(aggp, degt, W1, b1, w2p)

    zed2b = jnp.zeros((_NPAD, _D2), jnp.float32)
    src2 = src.reshape(_NW, _EPT // _K2, _K2)
    dst2 = dst.reshape(_NW, _EPT // _K2, _K2)
    agg2p = _seg_d2(h2, src2, dst2, zed2b)

    out = pl.pallas_call(
        _fin_body,
        out_shape=jax.ShapeDtypeStruct((_N, _NCLS), jnp.float32),
    )(agg2p, degt, b2)
    return out
